# Initial kernel scaffold; baseline (speedup 1.0000x reference)
import jax
import jax.numpy as jnp
from jax.experimental import pallas as pl


def kernel(feat_p3, feat_p4, feat_p5, stem_w, stem_b, obj_w, obj_b, box_w, box_b):
    raise NotImplementedError("write your pallas kernel here")


# speedup vs baseline: 62.2784x; 62.2784x over previous
import functools
import math

import jax
import jax.numpy as jnp
from jax.experimental import pallas as pl
from jax.experimental.pallas import tpu as pltpu

IMG = 512.0
PRE = 400
POST = 100
THR = 0.7
CLAMP = math.log(224.0 / 8.0)
C = 256
ROWS = 16


def _conv_body(x_ref, w9_ref, sb_ref, wh_ref, bh_ref, out_ref, *, H, W, B):
    i = pl.program_id(0)
    acc = jnp.zeros((B * ROWS * W, C), jnp.float32)
    for dy in range(3):
        for dx in range(3):
            xs = x_ref[:, pl.ds(i * ROWS + dy, ROWS), pl.ds(dx, W), :]
            acc = acc + jnp.dot(xs.reshape(B * ROWS * W, C), w9_ref[dy * 3 + dx],
                                preferred_element_type=jnp.float32)
    stem = jnp.maximum(acc + sb_ref[...], 0.0)
    head = jnp.dot(stem, wh_ref[...], preferred_element_type=jnp.float32) + bh_ref[...]
    out_ref[...] = head.reshape(B, ROWS, W, 16)


def _conv_heads(feat, w9, sb, wh, bh, H, W, B):
    xpad = jnp.pad(jnp.transpose(feat, (0, 2, 3, 1)), ((0, 0), (1, 1), (1, 1), (0, 0)))
    return pl.pallas_call(
        functools.partial(_conv_body, H=H, W=W, B=B),
        grid=(H // ROWS,),
        in_specs=[
            pl.BlockSpec((B, H + 2, W + 2, C), lambda i: (0, 0, 0, 0)),
            pl.BlockSpec((9, C, C), lambda i: (0, 0, 0)),
            pl.BlockSpec((1, C), lambda i: (0, 0)),
            pl.BlockSpec((C, 16), lambda i: (0, 0)),
            pl.BlockSpec((1, 16), lambda i: (0, 0)),
        ],
        out_specs=pl.BlockSpec((B, ROWS, W, 16), lambda i: (0, i, 0, 0)),
        out_shape=jax.ShapeDtypeStruct((B, H, W, 16), jnp.float32),
    )(xpad, w9, sb, wh, bh)


def _nms_body(d_ref, idx_ref, out_ref, iou_ref):
    idx = idx_ref[...]
    k6 = jax.lax.broadcasted_iota(jnp.int32, (6, 1), 0).astype(jnp.float32)
    lvl = jnp.floor(k6 * 0.5)
    stride = 8.0 * jnp.exp2(lvl)
    wgrid = 64.0 * jnp.exp2(-lvl)

    loc = jnp.floor(idx / 3.0)
    a = idx - 3.0 * loc
    row = jnp.floor(loc / wgrid)
    col = loc - wgrid * row
    cx = (col + 0.5) * stride
    cy = (row + 0.5) * stride
    area = (8.0 * stride) ** 2
    w0 = jnp.sqrt(area * 2.0)
    w1 = jnp.sqrt(area)
    w2 = jnp.sqrt(area * 0.5)
    aw0 = jnp.where(a < 0.5, w0, jnp.where(a < 1.5, w1, w2))
    ah0 = jnp.where(a < 0.5, area / w0, jnp.where(a < 1.5, area / w1, area / w2))
    a0 = cx - aw0 / 2.0
    a1 = cy - ah0 / 2.0
    a2 = cx + aw0 / 2.0
    a3 = cy + ah0 / 2.0

    aw = a2 - a0
    ah = a3 - a1
    acx = a0 + 0.5 * aw
    acy = a1 + 0.5 * ah
    d = d_ref[...]
    dw = jnp.minimum(d[:, :, 2], CLAMP)
    dh = jnp.minimum(d[:, :, 3], CLAMP)
    pcx = d[:, :, 0] * aw + acx
    pcy = d[:, :, 1] * ah + acy
    pw = jnp.exp(dw) * aw
    ph = jnp.exp(dh) * ah
    x1 = jnp.clip(pcx - 0.5 * pw, 0.0, IMG)
    y1 = jnp.clip(pcy - 0.5 * ph, 0.0, IMG)
    x2 = jnp.clip(pcx + 0.5 * pw, 0.0, IMG)
    y2 = jnp.clip(pcy + 0.5 * ph, 0.0, IMG)

    areas = (x2 - x1) * (y2 - y1)
    CH = 100
    for c in range(PRE // CH):
        sl = pl.ds(c * CH, CH)
        xx1 = jnp.maximum(x1[:, c * CH:(c + 1) * CH, None], x1[:, None, :])
        yy1 = jnp.maximum(y1[:, c * CH:(c + 1) * CH, None], y1[:, None, :])
        xx2 = jnp.minimum(x2[:, c * CH:(c + 1) * CH, None], x2[:, None, :])
        yy2 = jnp.minimum(y2[:, c * CH:(c + 1) * CH, None], y2[:, None, :])
        inter = jnp.clip(xx2 - xx1, 0.0) * jnp.clip(yy2 - yy1, 0.0)
        union = areas[:, c * CH:(c + 1) * CH, None] + areas[:, None, :] - inter
        iou_ref[:, sl, :] = inter / jnp.clip(union, 1e-8)

    jidx = jax.lax.broadcasted_iota(jnp.int32, (1, PRE), 1).astype(jnp.float32)

    def body(i, keepf):
        i_f = i.astype(jnp.float32)
        oh = (jidx == i_f).astype(jnp.float32)
        keep_i = jnp.sum(keepf * oh, axis=1, keepdims=True)
        iou_row = iou_ref[:, pl.ds(i, 1), :][:, 0, :]
        supf = ((iou_row > THR) & (jidx > i_f)).astype(jnp.float32) * keep_i
        return keepf * (1.0 - supf)

    kf = jax.lax.fori_loop(0, PRE, body, jnp.ones((6, PRE), jnp.float32))
    keep = kf > 0.5
    ii = jax.lax.broadcasted_iota(jnp.int32, (PRE, PRE), 0).astype(jnp.float32)
    jj = jax.lax.broadcasted_iota(jnp.int32, (PRE, PRE), 1).astype(jnp.float32)
    tri = (ii <= jj).astype(jnp.float32)
    cum = jnp.dot(kf, tri, preferred_element_type=jnp.float32)
    cumn = jnp.dot(1.0 - kf, tri, preferred_element_type=jnp.float32)
    nk = cum[:, PRE - 1:PRE]
    slot = jnp.where(keep, cum - 1.0, nk + cumn - 1.0)

    rr = jax.lax.broadcasted_iota(jnp.int32, (POST, PRE), 0).astype(jnp.float32)
    boxes4 = jnp.stack([x1, y1, x2, y2], axis=-1)
    for k in range(6):
        sel = (slot[k][None, :] == rr).astype(jnp.float32)
        out_ref[k, :, :] = jnp.dot(sel, boxes4[k], preferred_element_type=jnp.float32)


def _nms(d6, idx6):
    return pl.pallas_call(
        _nms_body,
        in_specs=[
            pl.BlockSpec((6, PRE, 4), lambda: (0, 0, 0)),
            pl.BlockSpec((6, PRE), lambda: (0, 0)),
        ],
        out_specs=pl.BlockSpec((6, POST, 4), lambda: (0, 0, 0)),
        out_shape=jax.ShapeDtypeStruct((6, POST, 4), jnp.float32),
        scratch_shapes=[pltpu.VMEM((6, PRE, PRE), jnp.float32)],
    )(d6, idx6)


def kernel(feat_p3, feat_p4, feat_p5, stem_w, stem_b, obj_w, obj_b, box_w, box_b):
    w9 = jnp.transpose(stem_w, (2, 3, 1, 0)).reshape(9, C, C)
    wh = jnp.concatenate([obj_w[:, :, 0, 0].T, box_w[:, :, 0, 0].T], axis=1)
    wh = jnp.pad(wh, ((0, 0), (0, 1)))
    bh = jnp.pad(jnp.concatenate([obj_b, box_b]), (0, 1)).reshape(1, 16)
    sb = stem_b.reshape(1, C)

    d_all, idx_all = [], []
    for feat, H in ((feat_p3, 64), (feat_p4, 32), (feat_p5, 16)):
        head = _conv_heads(feat, w9, sb, wh, bh, H, H, 2)
        logits = head[..., :3].reshape(2, -1)
        deltas = head[..., 3:15].reshape(2, -1, 4)
        _, idx = jax.lax.top_k(logits, PRE)
        d_all.append(jnp.take_along_axis(deltas, idx[..., None], axis=1))
        idx_all.append(idx)
    d6 = jnp.stack(d_all).reshape(6, PRE, 4)
    idx6 = jnp.stack(idx_all).reshape(6, PRE).astype(jnp.float32)
    out6 = _nms(d6, idx6)
    return out6.reshape(3, 2, POST, 4).transpose(1, 0, 2, 3).reshape(2, 3 * POST, 4)
